# Initial kernel scaffold; baseline (speedup 1.0000x reference)
import jax
import jax.numpy as jnp
from jax.experimental import pallas as pl


def kernel(x, edge_index, edge_type, batch, shape_emb, color_emb, pos_emb, W1, W1_root, b1, W2, W2_root, b2, cls_W, cls_b):
    raise NotImplementedError("write your pallas kernel here")


# speedup vs baseline: 5.7386x; 5.7386x over previous
import functools

import jax
import jax.numpy as jnp
from jax import lax
from jax.experimental import pallas as pl
from jax.experimental.pallas import tpu as pltpu
from jax.experimental.pallas import tpu_sc as plsc

N = 10000
E = 160000
RELS = 3
G = 128
HID = 256
IN1 = 192
NLB = 10

NSUB = 16
NCORE = 2
BATCH = 128
NB = 80
ET = NB * BATCH
EPAD = NSUB * ET

NP2 = 10112
ROWS = RELS * NP2
SHARE = ROWS // NSUB
CHUNK = 128
NCHUNK = 15
TAIL = SHARE - (NCHUNK - 1) * CHUNK
CNTW = 8

FS = 32
NS1 = IN1 // FS
NS2 = HID // FS

BN = 1000
NBLK = N // BN


def _sc_pass(c, p, h_slices, src_v, offs_v, rows_v, zeros_v, bounce_v,
             acc_sp, a_outs, row0, sem, cnt_stuff):
    for k in range(NCHUNK):
        nr = CHUNK if k < NCHUNK - 1 else TAIL
        pltpu.sync_copy(zeros_v.at[pl.ds(0, nr)],
                        acc_sp.at[pl.ds(row0 + k * CHUNK, nr)])
    if cnt_stuff is not None and p == 0:
        ones_v, zeros8_v, cnt_sp, cnt_out = cnt_stuff
        for k in range(NCHUNK):
            nr = CHUNK if k < NCHUNK - 1 else TAIL
            pltpu.sync_copy(zeros8_v.at[pl.ds(0, nr)],
                            cnt_sp.at[pl.ds(row0 + k * CHUNK, nr)])
    plsc.subcore_barrier()

    for cc in range(NCORE):
        @pl.when(c == cc)
        def _(cc=cc):
            hsrc = h_slices[2 * p + cc]

            def batch_body(b, carry):
                idx = src_v.at[b]
                pltpu.async_copy(hsrc.at[idx], rows_v, sem).wait()
                pltpu.sync_copy(rows_v, acc_sp.at[offs_v.at[b]], add=True)
                if cnt_stuff is not None and p == 0:
                    pltpu.sync_copy(cnt_stuff[0],
                                    cnt_sp.at[offs_v.at[b]], add=True)
                return carry

            lax.fori_loop(0, NB, batch_body, None)

    plsc.subcore_barrier()

    for cc in range(NCORE):
        @pl.when(c == cc)
        def _(cc=cc):
            a_out = a_outs[2 * p + cc]
            for k in range(NCHUNK):
                nr = CHUNK if k < NCHUNK - 1 else TAIL
                r0 = row0 + k * CHUNK
                pltpu.sync_copy(acc_sp.at[pl.ds(r0, nr)],
                                bounce_v.at[pl.ds(0, nr)])
                pltpu.sync_copy(bounce_v.at[pl.ds(0, nr)],
                                a_out.at[pl.ds(r0, nr)])
    if cnt_stuff is not None and p == 0:
        ones_v, zeros8_v, cnt_sp, cnt_out = cnt_stuff

        @pl.when(c == 0)
        def _():
            for k in range(NCHUNK):
                nr = CHUNK if k < NCHUNK - 1 else TAIL
                r0 = row0 + k * CHUNK
                pltpu.sync_copy(cnt_sp.at[pl.ds(r0, nr)],
                                zeros8_v.at[pl.ds(0, nr)])
                pltpu.sync_copy(zeros8_v.at[pl.ds(0, nr)],
                                cnt_out.at[pl.ds(r0, nr)])


def _make_sc1():
    mesh = plsc.VectorSubcoreMesh(core_axis_name="c", subcore_axis_name="s",
                                  num_cores=NCORE, num_subcores=NSUB)
    out_type = (
        [jax.ShapeDtypeStruct((ROWS, FS), jnp.float32) for _ in range(NS1)]
        + [jax.ShapeDtypeStruct((ROWS, CNTW), jnp.float32),
           jax.ShapeDtypeStruct((NSUB, NB, BATCH), jnp.int32)]
    )
    scratch = [
        pltpu.VMEM((NB, BATCH), jnp.int32),
        pltpu.VMEM((NB, BATCH), jnp.int32),
        pltpu.VMEM((NB, BATCH), jnp.int32),
        pltpu.VMEM((BATCH, FS), jnp.float32),
        pltpu.VMEM((BATCH, CNTW), jnp.float32),
        pltpu.VMEM((CHUNK, FS), jnp.float32),
        pltpu.VMEM((CHUNK, CNTW), jnp.float32),
        pltpu.VMEM((CHUNK, FS), jnp.float32),
        pltpu.VMEM_SHARED((ROWS, FS), jnp.float32),
        pltpu.VMEM_SHARED((ROWS, CNTW), jnp.float32),
        pltpu.SemaphoreType.DMA,
    ]

    @functools.partial(pl.kernel, out_type=out_type, mesh=mesh,
                       scratch_types=scratch,
                       compiler_params=pltpu.CompilerParams(
                           use_tc_tiling_on_sc=False))
    def sc1(*refs):
        h_slices = refs[:NS1]
        srcR, dstR, typeR, ones_h, zeros_h, zeros8_h = refs[NS1:NS1 + 6]
        a_outs = refs[NS1 + 6:2 * NS1 + 6]
        cnt_out, offs_out = refs[2 * NS1 + 6:2 * NS1 + 8]
        (src_v, offs_v, dst_v, rows_v, ones_v, zeros_v, zeros8_v, bounce_v,
         acc_sp, cnt_sp, sem) = refs[2 * NS1 + 8:]
        c = lax.axis_index("c")
        s = lax.axis_index("s")
        row0 = s * SHARE
        pltpu.sync_copy(srcR.at[s], src_v)
        pltpu.sync_copy(typeR.at[s], offs_v)
        pltpu.sync_copy(dstR.at[s], dst_v)
        pltpu.sync_copy(ones_h, ones_v)
        pltpu.sync_copy(zeros_h, zeros_v)
        pltpu.sync_copy(zeros8_h, zeros8_v)

        def offs_row(b, carry):
            for k in range(BATCH // 16):
                sl = pl.ds(k * 16, 16)
                offs_v[b, sl] = offs_v[b, sl] * NP2 + dst_v[b, sl]
            return carry

        lax.fori_loop(0, NB, offs_row, None)

        @pl.when(c == 0)
        def _():
            pltpu.sync_copy(offs_v, offs_out.at[s])

        cnt_stuff = (ones_v, zeros8_v, cnt_sp, cnt_out)
        for p in range(NS1 // 2):
            _sc_pass(c, p, h_slices, src_v, offs_v, rows_v, zeros_v,
                     bounce_v, acc_sp, a_outs, row0, sem,
                     cnt_stuff if p == 0 else None)

    return sc1


def _make_sc2():
    mesh = plsc.VectorSubcoreMesh(core_axis_name="c", subcore_axis_name="s",
                                  num_cores=NCORE, num_subcores=NSUB)
    out_type = [jax.ShapeDtypeStruct((ROWS, FS), jnp.float32)
                for _ in range(NS2)]
    scratch = [
        pltpu.VMEM((NB, BATCH), jnp.int32),
        pltpu.VMEM((NB, BATCH), jnp.int32),
        pltpu.VMEM((BATCH, FS), jnp.float32),
        pltpu.VMEM((CHUNK, FS), jnp.float32),
        pltpu.VMEM((CHUNK, FS), jnp.float32),
        pltpu.VMEM_SHARED((ROWS, FS), jnp.float32),
        pltpu.SemaphoreType.DMA,
    ]

    @functools.partial(pl.kernel, out_type=out_type, mesh=mesh,
                       scratch_types=scratch,
                       compiler_params=pltpu.CompilerParams(
                           use_tc_tiling_on_sc=False))
    def sc2(*refs):
        h_slices = refs[:NS2]
        srcR, offsR, zeros_h = refs[NS2:NS2 + 3]
        a_outs = refs[NS2 + 3:2 * NS2 + 3]
        (src_v, offs_v, rows_v, zeros_v, bounce_v, acc_sp,
         sem) = refs[2 * NS2 + 3:]
        c = lax.axis_index("c")
        s = lax.axis_index("s")
        row0 = s * SHARE
        pltpu.sync_copy(srcR.at[s], src_v)
        pltpu.sync_copy(offsR.at[s], offs_v)
        pltpu.sync_copy(zeros_h, zeros_v)

        for p in range(NS2 // 2):
            _sc_pass(c, p, h_slices, src_v, offs_v, rows_v, zeros_v,
                     bounce_v, acc_sp, a_outs, row0, sem, None)

    return sc2


def _embed_body(*refs):
    x_ref, se_ref, ce_ref, pe_ref = refs[:4]
    outs = refs[4:]
    xs = x_ref[:, 0:1]
    xc = x_ref[:, 1:2]
    xp = jnp.clip(x_ref[:, 2:3], 0, 63)
    ohs = (xs == lax.broadcasted_iota(jnp.int32, (BN, 16), 1)
           ).astype(jnp.float32)
    ohc = (xc == lax.broadcasted_iota(jnp.int32, (BN, 16), 1)
           ).astype(jnp.float32)
    ohp = (xp == lax.broadcasted_iota(jnp.int32, (BN, 64), 1)
           ).astype(jnp.float32)
    es = jnp.dot(ohs, se_ref[...], preferred_element_type=jnp.float32)
    ec = jnp.dot(ohc, ce_ref[...], preferred_element_type=jnp.float32)
    ep = jnp.dot(ohp, pe_ref[...], preferred_element_type=jnp.float32)
    feat = jnp.concatenate([es, ec, ep], axis=1)
    for j, o in enumerate(outs):
        o[...] = feat[:, j * FS:(j + 1) * FS]


def _l1_body(*refs):
    fs = refs[:NS1]
    a_s = refs[NS1:2 * NS1]
    cnt_ref, wr_ref, ws_ref, b_ref = refs[2 * NS1:2 * NS1 + 4]
    outs = refs[2 * NS1 + 4:]
    feat = jnp.concatenate([f[...] for f in fs], axis=1)
    acc = jnp.dot(feat, wr_ref[...], preferred_element_type=jnp.float32)
    acc = acc + b_ref[...]
    for r in range(RELS):
        ar = jnp.concatenate([a[r] for a in a_s], axis=1)
        inv = 1.0 / jnp.maximum(cnt_ref[r, :, 0:1], 1.0)
        acc = acc + jnp.dot(ar * inv, ws_ref[r],
                            preferred_element_type=jnp.float32)
    h = jnp.maximum(acc, 0.0)
    for j, o in enumerate(outs):
        o[...] = h[:, j * FS:(j + 1) * FS]


def _l2_body(*refs):
    hs = refs[:NS2]
    a_s = refs[NS2:2 * NS2]
    (cnt_ref, wr_ref, ws_ref, b_ref, bat_ref, cw_ref,
     cb_ref) = refs[2 * NS2:2 * NS2 + 7]
    out_ref, pool_acc, cnt_acc = refs[2 * NS2 + 7:]
    i = pl.program_id(0)

    @pl.when(i == 0)
    def _():
        pool_acc[...] = jnp.zeros_like(pool_acc)
        cnt_acc[...] = jnp.zeros_like(cnt_acc)

    hcat = jnp.concatenate([h[...] for h in hs], axis=1)
    acc = jnp.dot(hcat, wr_ref[...], preferred_element_type=jnp.float32)
    acc = acc + b_ref[...]
    for r in range(RELS):
        ar = jnp.concatenate([a[r] for a in a_s], axis=1)
        inv = 1.0 / jnp.maximum(cnt_ref[r, :, 0:1], 1.0)
        acc = acc + jnp.dot(ar * inv, ws_ref[r],
                            preferred_element_type=jnp.float32)
    h2v = jnp.maximum(acc, 0.0)

    bat = bat_ref[0]
    oh = (bat == lax.broadcasted_iota(jnp.int32, (G, BN), 0)
          ).astype(jnp.float32)
    pool_acc[...] = pool_acc[...] + jnp.dot(
        oh, h2v, preferred_element_type=jnp.float32)
    cnt_acc[...] = cnt_acc[...] + jnp.sum(oh, axis=1, keepdims=True)

    @pl.when(i == NBLK - 1)
    def _():
        hg = pool_acc[...] / jnp.maximum(cnt_acc[:, 0:1], 1.0)
        out_ref[...] = (jnp.dot(hg, cw_ref[...],
                                preferred_element_type=jnp.float32)
                        + cb_ref[...])


def _full(block):
    nd = len(block)
    return pl.BlockSpec(block, lambda i: (0,) * nd)


def _rowblk(block):
    nd = len(block)
    return pl.BlockSpec(block, lambda i: (i,) + (0,) * (nd - 1))


def _relblk(block):
    return pl.BlockSpec(block, lambda i: (0, i, 0))


_t1 = pl.pallas_call(
    _embed_body,
    grid=(NBLK,),
    in_specs=[_rowblk((BN, 3)), _full((16, 64)), _full((16, 64)),
              _full((64, 64))],
    out_specs=[_rowblk((BN, FS))] * NS1,
    out_shape=[jax.ShapeDtypeStruct((N, FS), jnp.float32)] * NS1,
)

_t2 = pl.pallas_call(
    _l1_body,
    grid=(NBLK,),
    in_specs=([_rowblk((BN, FS))] * NS1
              + [_relblk((RELS, BN, FS))] * NS1
              + [_relblk((RELS, BN, CNTW)), _full((IN1, HID)),
                 _full((RELS, IN1, HID)), _full((1, HID))]),
    out_specs=[_rowblk((BN, FS))] * NS2,
    out_shape=[jax.ShapeDtypeStruct((N, FS), jnp.float32)] * NS2,
)

_t3 = pl.pallas_call(
    _l2_body,
    grid=(NBLK,),
    in_specs=([_rowblk((BN, FS))] * NS2
              + [_relblk((RELS, BN, FS))] * NS2
              + [_relblk((RELS, BN, CNTW)), _full((HID, HID)),
                 _full((RELS, HID, HID)), _full((1, HID)),
                 pl.BlockSpec((1, 1, BN), lambda i: (i, 0, 0)),
                 _full((HID, NLB)), _full((1, NLB))]),
    out_specs=_full((G, NLB)),
    out_shape=jax.ShapeDtypeStruct((G, NLB), jnp.float32),
    scratch_shapes=[pltpu.VMEM((G, HID), jnp.float32),
                    pltpu.VMEM((G, 128), jnp.float32)],
)

_sc_cache = {}


def _get_sc():
    if "sc" not in _sc_cache:
        _sc_cache["sc"] = (_make_sc1(), _make_sc2())
    return _sc_cache["sc"]


def kernel(x, edge_index, edge_type, batch, shape_emb, color_emb, pos_emb,
           W1, W1_root, b1, W2, W2_root, b2, cls_W, cls_b):
    pad = EPAD - E
    src_p = jnp.concatenate(
        [edge_index[0], jnp.zeros((pad,), jnp.int32)]).reshape(NSUB, NB, BATCH)
    dst_p = jnp.concatenate(
        [edge_index[1], jnp.full((pad,), N, jnp.int32)]).reshape(NSUB, NB, BATCH)
    typ_p = jnp.concatenate(
        [edge_type, jnp.zeros((pad,), jnp.int32)]).reshape(NSUB, NB, BATCH)

    ones8 = jnp.ones((BATCH, CNTW), jnp.float32)
    zerosF = jnp.zeros((CHUNK, FS), jnp.float32)
    zeros8 = jnp.zeros((CHUNK, CNTW), jnp.float32)

    _sc1, _sc2 = _get_sc()
    f = _t1(x, shape_emb, color_emb, pos_emb)

    sc1_out = _sc1(*f, src_p, dst_p, typ_p, ones8, zerosF, zeros8)
    a = sc1_out[:NS1]
    cnt, offs = sc1_out[NS1], sc1_out[NS1 + 1]
    a_r = [ai.reshape(RELS, NP2, FS) for ai in a]
    cnt_r = cnt.reshape(RELS, NP2, CNTW)

    h = _t2(*f, *a_r, cnt_r, W1_root, W1, b1.reshape(1, HID))

    g = _sc2(*h, src_p, offs, zerosF)
    g_r = [gi.reshape(RELS, NP2, FS) for gi in g]

    bat3 = batch.reshape(NBLK, 1, BN)
    out = _t3(*h, *g_r, cnt_r, W2_root, W2, b2.reshape(1, HID),
              bat3, cls_W, cls_b.reshape(1, NLB))
    return out


# speedup vs baseline: 7.1734x; 1.2500x over previous
import functools

import jax
import jax.numpy as jnp
from jax import lax
from jax.experimental import pallas as pl
from jax.experimental.pallas import tpu as pltpu
from jax.experimental.pallas import tpu_sc as plsc

N = 10000
E = 160000
RELS = 3
G = 128
HID = 256
IN1 = 192
NLB = 10

NSUB = 16
NCORE = 2
BATCH = 128
NB = 80
ET = NB * BATCH
EPAD = NSUB * ET

NP2 = 10112
ROWS = RELS * NP2
SHARE = ROWS // NSUB
CHUNK = 128
NCHUNK = 15
TAIL = SHARE - (NCHUNK - 1) * CHUNK
CNTW = 8

FS = 32
NS1 = IN1 // FS
NS2 = HID // FS

BN = 1000
NBLK = N // BN


def _sc_pass(c, p, h_slices, src_v, offs_v, rows_v, zeros_v, bounce_v,
             acc_sp, a_outs, row0, sem, cnt_stuff):
    for k in range(NCHUNK):
        nr = CHUNK if k < NCHUNK - 1 else TAIL
        pltpu.sync_copy(zeros_v.at[pl.ds(0, nr)],
                        acc_sp.at[pl.ds(row0 + k * CHUNK, nr)])
    if cnt_stuff is not None and p == 0:
        ones_v, zeros8_v, cnt_sp, cnt_out = cnt_stuff
        for k in range(NCHUNK):
            nr = CHUNK if k < NCHUNK - 1 else TAIL
            pltpu.sync_copy(zeros8_v.at[pl.ds(0, nr)],
                            cnt_sp.at[pl.ds(row0 + k * CHUNK, nr)])
    plsc.subcore_barrier()

    rows_a, rows_b = rows_v
    sem_a, sem_b = sem
    for cc in range(NCORE):
        @pl.when(c == cc)
        def _(cc=cc):
            hsrc = h_slices[2 * p + cc]

            def issue(b, buf, sm):
                pltpu.async_copy(hsrc.at[src_v.at[b]], buf, sm)

            def waitg(buf, sm):
                pltpu.make_async_copy(hsrc.at[src_v.at[0]], buf, sm).wait()

            def scat(b, buf):
                pltpu.sync_copy(buf, acc_sp.at[offs_v.at[b]], add=True)
                if cnt_stuff is not None and p == 0:
                    pltpu.sync_copy(cnt_stuff[0],
                                    cnt_sp.at[offs_v.at[b]], add=True)

            issue(0, rows_a, sem_a)

            def batch_body(t, carry):
                b0 = 2 * t
                issue(b0 + 1, rows_b, sem_b)
                waitg(rows_a, sem_a)
                scat(b0, rows_a)

                @pl.when(b0 + 2 < NB)
                def _():
                    issue(b0 + 2, rows_a, sem_a)

                waitg(rows_b, sem_b)
                scat(b0 + 1, rows_b)
                return carry

            lax.fori_loop(0, NB // 2, batch_body, None)

    plsc.subcore_barrier()

    for cc in range(NCORE):
        @pl.when(c == cc)
        def _(cc=cc):
            a_out = a_outs[2 * p + cc]
            for k in range(NCHUNK):
                nr = CHUNK if k < NCHUNK - 1 else TAIL
                r0 = row0 + k * CHUNK
                pltpu.sync_copy(acc_sp.at[pl.ds(r0, nr)],
                                bounce_v.at[pl.ds(0, nr)])
                pltpu.sync_copy(bounce_v.at[pl.ds(0, nr)],
                                a_out.at[pl.ds(r0, nr)])
    if cnt_stuff is not None and p == 0:
        ones_v, zeros8_v, cnt_sp, cnt_out = cnt_stuff

        @pl.when(c == 0)
        def _():
            for k in range(NCHUNK):
                nr = CHUNK if k < NCHUNK - 1 else TAIL
                r0 = row0 + k * CHUNK
                pltpu.sync_copy(cnt_sp.at[pl.ds(r0, nr)],
                                zeros8_v.at[pl.ds(0, nr)])
                pltpu.sync_copy(zeros8_v.at[pl.ds(0, nr)],
                                cnt_out.at[pl.ds(r0, nr)])


def _make_sc1():
    mesh = plsc.VectorSubcoreMesh(core_axis_name="c", subcore_axis_name="s",
                                  num_cores=NCORE, num_subcores=NSUB)
    out_type = (
        [jax.ShapeDtypeStruct((ROWS, FS), jnp.float32) for _ in range(NS1)]
        + [jax.ShapeDtypeStruct((ROWS, CNTW), jnp.float32),
           jax.ShapeDtypeStruct((NSUB, NB, BATCH), jnp.int32)]
    )
    scratch = [
        pltpu.VMEM((NB, BATCH), jnp.int32),
        pltpu.VMEM((NB, BATCH), jnp.int32),
        pltpu.VMEM((NB, BATCH), jnp.int32),
        pltpu.VMEM((BATCH, FS), jnp.float32),
        pltpu.VMEM((BATCH, FS), jnp.float32),
        pltpu.VMEM((BATCH, CNTW), jnp.float32),
        pltpu.VMEM((CHUNK, FS), jnp.float32),
        pltpu.VMEM((CHUNK, CNTW), jnp.float32),
        pltpu.VMEM((CHUNK, FS), jnp.float32),
        pltpu.VMEM_SHARED((ROWS, FS), jnp.float32),
        pltpu.VMEM_SHARED((ROWS, CNTW), jnp.float32),
        pltpu.SemaphoreType.DMA,
        pltpu.SemaphoreType.DMA,
    ]

    @functools.partial(pl.kernel, out_type=out_type, mesh=mesh,
                       scratch_types=scratch,
                       compiler_params=pltpu.CompilerParams(
                           use_tc_tiling_on_sc=False))
    def sc1(*refs):
        h_slices = refs[:NS1]
        srcR, dstR, typeR, ones_h, zeros_h, zeros8_h = refs[NS1:NS1 + 6]
        a_outs = refs[NS1 + 6:2 * NS1 + 6]
        cnt_out, offs_out = refs[2 * NS1 + 6:2 * NS1 + 8]
        (src_v, offs_v, dst_v, rows_a, rows_b, ones_v, zeros_v, zeros8_v,
         bounce_v, acc_sp, cnt_sp, sem_a, sem_b) = refs[2 * NS1 + 8:]
        rows_v = (rows_a, rows_b)
        sem = (sem_a, sem_b)
        c = lax.axis_index("c")
        s = lax.axis_index("s")
        row0 = s * SHARE
        pltpu.sync_copy(srcR.at[s], src_v)
        pltpu.sync_copy(typeR.at[s], offs_v)
        pltpu.sync_copy(dstR.at[s], dst_v)
        pltpu.sync_copy(ones_h, ones_v)
        pltpu.sync_copy(zeros_h, zeros_v)
        pltpu.sync_copy(zeros8_h, zeros8_v)

        def offs_row(b, carry):
            for k in range(BATCH // 16):
                sl = pl.ds(k * 16, 16)
                offs_v[b, sl] = offs_v[b, sl] * NP2 + dst_v[b, sl]
            return carry

        lax.fori_loop(0, NB, offs_row, None)

        @pl.when(c == 0)
        def _():
            pltpu.sync_copy(offs_v, offs_out.at[s])

        cnt_stuff = (ones_v, zeros8_v, cnt_sp, cnt_out)
        for p in range(NS1 // 2):
            _sc_pass(c, p, h_slices, src_v, offs_v, rows_v, zeros_v,
                     bounce_v, acc_sp, a_outs, row0, sem,
                     cnt_stuff if p == 0 else None)

    return sc1


def _make_sc2():
    mesh = plsc.VectorSubcoreMesh(core_axis_name="c", subcore_axis_name="s",
                                  num_cores=NCORE, num_subcores=NSUB)
    out_type = [jax.ShapeDtypeStruct((ROWS, FS), jnp.float32)
                for _ in range(NS2)]
    scratch = [
        pltpu.VMEM((NB, BATCH), jnp.int32),
        pltpu.VMEM((NB, BATCH), jnp.int32),
        pltpu.VMEM((BATCH, FS), jnp.float32),
        pltpu.VMEM((BATCH, FS), jnp.float32),
        pltpu.VMEM((CHUNK, FS), jnp.float32),
        pltpu.VMEM((CHUNK, FS), jnp.float32),
        pltpu.VMEM_SHARED((ROWS, FS), jnp.float32),
        pltpu.SemaphoreType.DMA,
        pltpu.SemaphoreType.DMA,
    ]

    @functools.partial(pl.kernel, out_type=out_type, mesh=mesh,
                       scratch_types=scratch,
                       compiler_params=pltpu.CompilerParams(
                           use_tc_tiling_on_sc=False))
    def sc2(*refs):
        h_slices = refs[:NS2]
        srcR, offsR, zeros_h = refs[NS2:NS2 + 3]
        a_outs = refs[NS2 + 3:2 * NS2 + 3]
        (src_v, offs_v, rows_a, rows_b, zeros_v, bounce_v, acc_sp,
         sem_a, sem_b) = refs[2 * NS2 + 3:]
        rows_v = (rows_a, rows_b)
        sem = (sem_a, sem_b)
        c = lax.axis_index("c")
        s = lax.axis_index("s")
        row0 = s * SHARE
        pltpu.sync_copy(srcR.at[s], src_v)
        pltpu.sync_copy(offsR.at[s], offs_v)
        pltpu.sync_copy(zeros_h, zeros_v)

        for p in range(NS2 // 2):
            _sc_pass(c, p, h_slices, src_v, offs_v, rows_v, zeros_v,
                     bounce_v, acc_sp, a_outs, row0, sem, None)

    return sc2


def _embed_body(*refs):
    x_ref, se_ref, ce_ref, pe_ref = refs[:4]
    outs = refs[4:]
    xs = x_ref[:, 0:1]
    xc = x_ref[:, 1:2]
    xp = jnp.clip(x_ref[:, 2:3], 0, 63)
    ohs = (xs == lax.broadcasted_iota(jnp.int32, (BN, 16), 1)
           ).astype(jnp.float32)
    ohc = (xc == lax.broadcasted_iota(jnp.int32, (BN, 16), 1)
           ).astype(jnp.float32)
    ohp = (xp == lax.broadcasted_iota(jnp.int32, (BN, 64), 1)
           ).astype(jnp.float32)
    es = jnp.dot(ohs, se_ref[...], preferred_element_type=jnp.float32)
    ec = jnp.dot(ohc, ce_ref[...], preferred_element_type=jnp.float32)
    ep = jnp.dot(ohp, pe_ref[...], preferred_element_type=jnp.float32)
    feat = jnp.concatenate([es, ec, ep], axis=1)
    for j, o in enumerate(outs):
        o[...] = feat[:, j * FS:(j + 1) * FS]


def _root_body(nsl, *refs):
    fs = refs[:nsl]
    wr_ref, b_ref, out_ref = refs[nsl:]
    feat = jnp.concatenate([f[...] for f in fs], axis=1)
    out_ref[...] = (jnp.dot(feat, wr_ref[...],
                            preferred_element_type=jnp.float32)
                    + b_ref[...])


def _l1_body(*refs):
    base_ref = refs[0]
    a_s = refs[1:1 + NS1]
    cnt_ref, ws_ref = refs[1 + NS1:3 + NS1]
    outs = refs[3 + NS1:]
    acc = base_ref[...]
    for r in range(RELS):
        ar = jnp.concatenate([a[r] for a in a_s], axis=1)
        inv = 1.0 / jnp.maximum(cnt_ref[r, :, 0:1], 1.0)
        acc = acc + jnp.dot(ar * inv, ws_ref[r],
                            preferred_element_type=jnp.float32)
    h = jnp.maximum(acc, 0.0)
    for j, o in enumerate(outs):
        o[...] = h[:, j * FS:(j + 1) * FS]


def _l2_body(*refs):
    base_ref = refs[0]
    a_s = refs[1:1 + NS2]
    (cnt_ref, ws_ref, bat_ref, cw_ref,
     cb_ref) = refs[1 + NS2:6 + NS2]
    out_ref, pool_acc, cnt_acc = refs[6 + NS2:]
    i = pl.program_id(0)

    @pl.when(i == 0)
    def _():
        pool_acc[...] = jnp.zeros_like(pool_acc)
        cnt_acc[...] = jnp.zeros_like(cnt_acc)

    acc = base_ref[...]
    for r in range(RELS):
        ar = jnp.concatenate([a[r] for a in a_s], axis=1)
        inv = 1.0 / jnp.maximum(cnt_ref[r, :, 0:1], 1.0)
        acc = acc + jnp.dot(ar * inv, ws_ref[r],
                            preferred_element_type=jnp.float32)
    h2v = jnp.maximum(acc, 0.0)

    bat = bat_ref[0]
    oh = (bat == lax.broadcasted_iota(jnp.int32, (G, BN), 0)
          ).astype(jnp.float32)
    pool_acc[...] = pool_acc[...] + jnp.dot(
        oh, h2v, preferred_element_type=jnp.float32)
    cnt_acc[...] = cnt_acc[...] + jnp.sum(oh, axis=1, keepdims=True)

    @pl.when(i == NBLK - 1)
    def _():
        hg = pool_acc[...] / jnp.maximum(cnt_acc[:, 0:1], 1.0)
        out_ref[...] = (jnp.dot(hg, cw_ref[...],
                                preferred_element_type=jnp.float32)
                        + cb_ref[...])


def _full(block):
    nd = len(block)
    return pl.BlockSpec(block, lambda i: (0,) * nd)


def _rowblk(block):
    nd = len(block)
    return pl.BlockSpec(block, lambda i: (i,) + (0,) * (nd - 1))


def _relblk(block):
    return pl.BlockSpec(block, lambda i: (0, i, 0))


_t1 = pl.pallas_call(
    _embed_body,
    grid=(NBLK,),
    in_specs=[_rowblk((BN, 3)), _full((16, 64)), _full((16, 64)),
              _full((64, 64))],
    out_specs=[_rowblk((BN, FS))] * NS1,
    out_shape=[jax.ShapeDtypeStruct((N, FS), jnp.float32)] * NS1,
)

_t2a = pl.pallas_call(
    functools.partial(_root_body, NS1),
    grid=(NBLK,),
    in_specs=[_rowblk((BN, FS))] * NS1 + [_full((IN1, HID)),
                                          _full((1, HID))],
    out_specs=_rowblk((BN, HID)),
    out_shape=jax.ShapeDtypeStruct((N, HID), jnp.float32),
)

_t3a = pl.pallas_call(
    functools.partial(_root_body, NS2),
    grid=(NBLK,),
    in_specs=[_rowblk((BN, FS))] * NS2 + [_full((HID, HID)),
                                          _full((1, HID))],
    out_specs=_rowblk((BN, HID)),
    out_shape=jax.ShapeDtypeStruct((N, HID), jnp.float32),
)

_t2 = pl.pallas_call(
    _l1_body,
    grid=(NBLK,),
    in_specs=([_rowblk((BN, HID))]
              + [_relblk((RELS, BN, FS))] * NS1
              + [_relblk((RELS, BN, CNTW)), _full((RELS, IN1, HID))]),
    out_specs=[_rowblk((BN, FS))] * NS2,
    out_shape=[jax.ShapeDtypeStruct((N, FS), jnp.float32)] * NS2,
)

_t3 = pl.pallas_call(
    _l2_body,
    grid=(NBLK,),
    in_specs=([_rowblk((BN, HID))]
              + [_relblk((RELS, BN, FS))] * NS2
              + [_relblk((RELS, BN, CNTW)), _full((RELS, HID, HID)),
                 pl.BlockSpec((1, 1, BN), lambda i: (i, 0, 0)),
                 _full((HID, NLB)), _full((1, NLB))]),
    out_specs=_full((G, NLB)),
    out_shape=jax.ShapeDtypeStruct((G, NLB), jnp.float32),
    scratch_shapes=[pltpu.VMEM((G, HID), jnp.float32),
                    pltpu.VMEM((G, 128), jnp.float32)],
)

_sc_cache = {}


def _get_sc():
    if "sc" not in _sc_cache:
        _sc_cache["sc"] = (_make_sc1(), _make_sc2())
    return _sc_cache["sc"]


def kernel(x, edge_index, edge_type, batch, shape_emb, color_emb, pos_emb,
           W1, W1_root, b1, W2, W2_root, b2, cls_W, cls_b):
    pad = EPAD - E
    src_p = jnp.concatenate(
        [edge_index[0], jnp.zeros((pad,), jnp.int32)]).reshape(NSUB, NB, BATCH)
    dst_p = jnp.concatenate(
        [edge_index[1], jnp.full((pad,), N, jnp.int32)]).reshape(NSUB, NB, BATCH)
    typ_p = jnp.concatenate(
        [edge_type, jnp.zeros((pad,), jnp.int32)]).reshape(NSUB, NB, BATCH)

    ones8 = jnp.ones((BATCH, CNTW), jnp.float32)
    zerosF = jnp.zeros((CHUNK, FS), jnp.float32)
    zeros8 = jnp.zeros((CHUNK, CNTW), jnp.float32)

    _sc1, _sc2 = _get_sc()
    f = _t1(x, shape_emb, color_emb, pos_emb)

    sc1_out = _sc1(*f, src_p, dst_p, typ_p, ones8, zerosF, zeros8)
    base1 = _t2a(*f, W1_root, b1.reshape(1, HID))
    a = sc1_out[:NS1]
    cnt, offs = sc1_out[NS1], sc1_out[NS1 + 1]
    a_r = [ai.reshape(RELS, NP2, FS) for ai in a]
    cnt_r = cnt.reshape(RELS, NP2, CNTW)

    h = _t2(base1, *a_r, cnt_r, W1)

    g = _sc2(*h, src_p, offs, zerosF)
    base2 = _t3a(*h, W2_root, b2.reshape(1, HID))
    g_r = [gi.reshape(RELS, NP2, FS) for gi in g]

    bat3 = batch.reshape(NBLK, 1, BN)
    out = _t3(base2, *g_r, cnt_r, W2, bat3, cls_W, cls_b.reshape(1, NLB))
    return out


# speedup vs baseline: 7.2961x; 1.0171x over previous
import functools

import jax
import jax.numpy as jnp
from jax import lax
from jax.experimental import pallas as pl
from jax.experimental.pallas import tpu as pltpu
from jax.experimental.pallas import tpu_sc as plsc

N = 10000
E = 160000
RELS = 3
G = 128
HID = 256
IN1 = 192
NLB = 10

NSUB = 16
NCORE = 2
BATCH = 128
NB = 80
ET = NB * BATCH
EPAD = NSUB * ET

NP2 = 10112
ROWS = RELS * NP2
SHARE = ROWS // NSUB
CHUNK = 128
NCHUNK = 15
TAIL = SHARE - (NCHUNK - 1) * CHUNK
CNTW = 8

FS = 32
NS1 = IN1 // FS
NS2 = HID // FS

BN = 1000
NBLK = N // BN
BNR = 2000
NBLKR = N // BNR


def _sc_pass(c, p, h_slices, src_v, offs_v, rows_v, zeros_v, bounce_v,
             acc_sp, a_outs, row0, sem, cnt_stuff):
    for k in range(NCHUNK):
        nr = CHUNK if k < NCHUNK - 1 else TAIL
        pltpu.sync_copy(zeros_v.at[pl.ds(0, nr)],
                        acc_sp.at[pl.ds(row0 + k * CHUNK, nr)])
    if cnt_stuff is not None and p == 0:
        ones_v, zeros8_v, cnt_sp, cnt_out = cnt_stuff
        for k in range(NCHUNK):
            nr = CHUNK if k < NCHUNK - 1 else TAIL
            pltpu.sync_copy(zeros8_v.at[pl.ds(0, nr)],
                            cnt_sp.at[pl.ds(row0 + k * CHUNK, nr)])
    plsc.subcore_barrier()

    rows_a, rows_b = rows_v
    sem_a, sem_b = sem
    for cc in range(NCORE):
        @pl.when(c == cc)
        def _(cc=cc):
            hsrc = h_slices[2 * p + cc]

            def issue(b, buf, sm):
                pltpu.async_copy(hsrc.at[src_v.at[b]], buf, sm)

            def waitg(buf, sm):
                pltpu.make_async_copy(hsrc.at[src_v.at[0]], buf, sm).wait()

            def scat(b, buf):
                pltpu.sync_copy(buf, acc_sp.at[offs_v.at[b]], add=True)
                if cnt_stuff is not None and p == 0:
                    pltpu.sync_copy(cnt_stuff[0],
                                    cnt_sp.at[offs_v.at[b]], add=True)

            issue(0, rows_a, sem_a)

            def batch_body(t, carry):
                b0 = 2 * t
                issue(b0 + 1, rows_b, sem_b)
                waitg(rows_a, sem_a)
                scat(b0, rows_a)

                @pl.when(b0 + 2 < NB)
                def _():
                    issue(b0 + 2, rows_a, sem_a)

                waitg(rows_b, sem_b)
                scat(b0 + 1, rows_b)
                return carry

            lax.fori_loop(0, NB // 2, batch_body, None)

    plsc.subcore_barrier()

    for cc in range(NCORE):
        @pl.when(c == cc)
        def _(cc=cc):
            a_out = a_outs[2 * p + cc]
            pltpu.sync_copy(acc_sp.at[pl.ds(row0, SHARE)],
                            a_out.at[pl.ds(row0, SHARE)])
    if cnt_stuff is not None and p == 0:
        ones_v, zeros8_v, cnt_sp, cnt_out = cnt_stuff

        @pl.when(c == 0)
        def _():
            pltpu.sync_copy(cnt_sp.at[pl.ds(row0, SHARE)],
                            cnt_out.at[pl.ds(row0, SHARE)])


def _make_sc1():
    mesh = plsc.VectorSubcoreMesh(core_axis_name="c", subcore_axis_name="s",
                                  num_cores=NCORE, num_subcores=NSUB)
    out_type = (
        [jax.ShapeDtypeStruct((ROWS, FS), jnp.float32) for _ in range(NS1)]
        + [jax.ShapeDtypeStruct((ROWS, CNTW), jnp.float32),
           jax.ShapeDtypeStruct((NSUB, NB, BATCH), jnp.int32)]
    )
    scratch = [
        pltpu.VMEM((NB, BATCH), jnp.int32),
        pltpu.VMEM((NB, BATCH), jnp.int32),
        pltpu.VMEM((NB, BATCH), jnp.int32),
        pltpu.VMEM((BATCH, FS), jnp.float32),
        pltpu.VMEM((BATCH, FS), jnp.float32),
        pltpu.VMEM((BATCH, CNTW), jnp.float32),
        pltpu.VMEM((CHUNK, FS), jnp.float32),
        pltpu.VMEM((CHUNK, CNTW), jnp.float32),
        pltpu.VMEM((CHUNK, FS), jnp.float32),
        pltpu.VMEM_SHARED((ROWS, FS), jnp.float32),
        pltpu.VMEM_SHARED((ROWS, CNTW), jnp.float32),
        pltpu.SemaphoreType.DMA,
        pltpu.SemaphoreType.DMA,
    ]

    @functools.partial(pl.kernel, out_type=out_type, mesh=mesh,
                       scratch_types=scratch,
                       compiler_params=pltpu.CompilerParams(
                           use_tc_tiling_on_sc=False))
    def sc1(*refs):
        h_slices = refs[:NS1]
        srcR, dstR, typeR, ones_h, zeros_h, zeros8_h = refs[NS1:NS1 + 6]
        a_outs = refs[NS1 + 6:2 * NS1 + 6]
        cnt_out, offs_out = refs[2 * NS1 + 6:2 * NS1 + 8]
        (src_v, offs_v, dst_v, rows_a, rows_b, ones_v, zeros_v, zeros8_v,
         bounce_v, acc_sp, cnt_sp, sem_a, sem_b) = refs[2 * NS1 + 8:]
        rows_v = (rows_a, rows_b)
        sem = (sem_a, sem_b)
        c = lax.axis_index("c")
        s = lax.axis_index("s")
        row0 = s * SHARE
        pltpu.sync_copy(srcR.at[s], src_v)
        pltpu.sync_copy(typeR.at[s], offs_v)
        pltpu.sync_copy(dstR.at[s], dst_v)
        pltpu.sync_copy(ones_h, ones_v)
        pltpu.sync_copy(zeros_h, zeros_v)
        pltpu.sync_copy(zeros8_h, zeros8_v)

        def offs_row(b, carry):
            for k in range(BATCH // 16):
                sl = pl.ds(k * 16, 16)
                offs_v[b, sl] = offs_v[b, sl] * NP2 + dst_v[b, sl]
            return carry

        lax.fori_loop(0, NB, offs_row, None)

        @pl.when(c == 0)
        def _():
            pltpu.sync_copy(offs_v, offs_out.at[s])

        cnt_stuff = (ones_v, zeros8_v, cnt_sp, cnt_out)
        for p in range(NS1 // 2):
            _sc_pass(c, p, h_slices, src_v, offs_v, rows_v, zeros_v,
                     bounce_v, acc_sp, a_outs, row0, sem,
                     cnt_stuff if p == 0 else None)

    return sc1


def _make_sc2():
    mesh = plsc.VectorSubcoreMesh(core_axis_name="c", subcore_axis_name="s",
                                  num_cores=NCORE, num_subcores=NSUB)
    out_type = [jax.ShapeDtypeStruct((ROWS, FS), jnp.float32)
                for _ in range(NS2)]
    scratch = [
        pltpu.VMEM((NB, BATCH), jnp.int32),
        pltpu.VMEM((NB, BATCH), jnp.int32),
        pltpu.VMEM((BATCH, FS), jnp.float32),
        pltpu.VMEM((BATCH, FS), jnp.float32),
        pltpu.VMEM((CHUNK, FS), jnp.float32),
        pltpu.VMEM((CHUNK, FS), jnp.float32),
        pltpu.VMEM_SHARED((ROWS, FS), jnp.float32),
        pltpu.SemaphoreType.DMA,
        pltpu.SemaphoreType.DMA,
    ]

    @functools.partial(pl.kernel, out_type=out_type, mesh=mesh,
                       scratch_types=scratch,
                       compiler_params=pltpu.CompilerParams(
                           use_tc_tiling_on_sc=False))
    def sc2(*refs):
        h_slices = refs[:NS2]
        srcR, offsR, zeros_h = refs[NS2:NS2 + 3]
        a_outs = refs[NS2 + 3:2 * NS2 + 3]
        (src_v, offs_v, rows_a, rows_b, zeros_v, bounce_v, acc_sp,
         sem_a, sem_b) = refs[2 * NS2 + 3:]
        rows_v = (rows_a, rows_b)
        sem = (sem_a, sem_b)
        c = lax.axis_index("c")
        s = lax.axis_index("s")
        row0 = s * SHARE
        pltpu.sync_copy(srcR.at[s], src_v)
        pltpu.sync_copy(offsR.at[s], offs_v)
        pltpu.sync_copy(zeros_h, zeros_v)

        for p in range(NS2 // 2):
            _sc_pass(c, p, h_slices, src_v, offs_v, rows_v, zeros_v,
                     bounce_v, acc_sp, a_outs, row0, sem, None)

    return sc2


def _embed_body(*refs):
    x_ref, se_ref, ce_ref, pe_ref = refs[:4]
    outs = refs[4:]
    xs = x_ref[:, 0:1]
    xc = x_ref[:, 1:2]
    xp = jnp.clip(x_ref[:, 2:3], 0, 63)
    ohs = (xs == lax.broadcasted_iota(jnp.int32, (BN, 16), 1)
           ).astype(jnp.float32)
    ohc = (xc == lax.broadcasted_iota(jnp.int32, (BN, 16), 1)
           ).astype(jnp.float32)
    ohp = (xp == lax.broadcasted_iota(jnp.int32, (BN, 64), 1)
           ).astype(jnp.float32)
    es = jnp.dot(ohs, se_ref[...], preferred_element_type=jnp.float32)
    ec = jnp.dot(ohc, ce_ref[...], preferred_element_type=jnp.float32)
    ep = jnp.dot(ohp, pe_ref[...], preferred_element_type=jnp.float32)
    feat = jnp.concatenate([es, ec, ep], axis=1)
    for j, o in enumerate(outs):
        o[...] = feat[:, j * FS:(j + 1) * FS]


def _root_body(nsl, *refs):
    fs = refs[:nsl]
    wr_ref, b_ref, out_ref = refs[nsl:]
    feat = jnp.concatenate([f[...] for f in fs], axis=1)
    out_ref[...] = (jnp.dot(feat, wr_ref[...],
                            preferred_element_type=jnp.float32)
                    + b_ref[...])


def _l1_body(*refs):
    base_ref = refs[0]
    a_s = refs[1:1 + NS1]
    cnt_ref, ws_ref = refs[1 + NS1:3 + NS1]
    outs = refs[3 + NS1:]
    acc = base_ref[...]
    for r in range(RELS):
        ar = jnp.concatenate([a[r] for a in a_s], axis=1)
        inv = 1.0 / jnp.maximum(cnt_ref[r, :, 0:1], 1.0)
        acc = acc + jnp.dot(ar * inv, ws_ref[r],
                            preferred_element_type=jnp.float32)
    h = jnp.maximum(acc, 0.0)
    for j, o in enumerate(outs):
        o[...] = h[:, j * FS:(j + 1) * FS]


def _l2_body(*refs):
    base_ref = refs[0]
    a_s = refs[1:1 + NS2]
    (cnt_ref, ws_ref, bat_ref, cw_ref,
     cb_ref) = refs[1 + NS2:6 + NS2]
    out_ref, pool_acc, cnt_acc = refs[6 + NS2:]
    i = pl.program_id(0)

    @pl.when(i == 0)
    def _():
        pool_acc[...] = jnp.zeros_like(pool_acc)
        cnt_acc[...] = jnp.zeros_like(cnt_acc)

    acc = base_ref[...]
    for r in range(RELS):
        ar = jnp.concatenate([a[r] for a in a_s], axis=1)
        inv = 1.0 / jnp.maximum(cnt_ref[r, :, 0:1], 1.0)
        acc = acc + jnp.dot(ar * inv, ws_ref[r],
                            preferred_element_type=jnp.float32)
    h2v = jnp.maximum(acc, 0.0)

    bat = bat_ref[0]
    oh = (bat == lax.broadcasted_iota(jnp.int32, (G, BN), 0)
          ).astype(jnp.float32)
    pool_acc[...] = pool_acc[...] + jnp.dot(
        oh, h2v, preferred_element_type=jnp.float32)
    cnt_acc[...] = cnt_acc[...] + jnp.sum(oh, axis=1, keepdims=True)

    @pl.when(i == NBLK - 1)
    def _():
        hg = pool_acc[...] / jnp.maximum(cnt_acc[:, 0:1], 1.0)
        out_ref[...] = (jnp.dot(hg, cw_ref[...],
                                preferred_element_type=jnp.float32)
                        + cb_ref[...])


def _full(block):
    nd = len(block)
    return pl.BlockSpec(block, lambda i: (0,) * nd)


def _rowblk(block):
    nd = len(block)
    return pl.BlockSpec(block, lambda i: (i,) + (0,) * (nd - 1))


def _relblk(block):
    return pl.BlockSpec(block, lambda i: (0, i, 0))


_t1 = pl.pallas_call(
    _embed_body,
    grid=(NBLK,),
    in_specs=[_rowblk((BN, 3)), _full((16, 64)), _full((16, 64)),
              _full((64, 64))],
    out_specs=[_rowblk((BN, FS))] * NS1,
    out_shape=[jax.ShapeDtypeStruct((N, FS), jnp.float32)] * NS1,
)

_t2a = pl.pallas_call(
    functools.partial(_root_body, NS1),
    grid=(NBLKR,),
    in_specs=[_rowblk((BNR, FS))] * NS1 + [_full((IN1, HID)),
                                           _full((1, HID))],
    out_specs=_rowblk((BNR, HID)),
    out_shape=jax.ShapeDtypeStruct((N, HID), jnp.float32),
)

_t3a = pl.pallas_call(
    functools.partial(_root_body, NS2),
    grid=(NBLKR,),
    in_specs=[_rowblk((BNR, FS))] * NS2 + [_full((HID, HID)),
                                           _full((1, HID))],
    out_specs=_rowblk((BNR, HID)),
    out_shape=jax.ShapeDtypeStruct((N, HID), jnp.float32),
)

_t2 = pl.pallas_call(
    _l1_body,
    grid=(NBLK,),
    in_specs=([_rowblk((BN, HID))]
              + [_relblk((RELS, BN, FS))] * NS1
              + [_relblk((RELS, BN, CNTW)), _full((RELS, IN1, HID))]),
    out_specs=[_rowblk((BN, FS))] * NS2,
    out_shape=[jax.ShapeDtypeStruct((N, FS), jnp.float32)] * NS2,
)

_t3 = pl.pallas_call(
    _l2_body,
    grid=(NBLK,),
    in_specs=([_rowblk((BN, HID))]
              + [_relblk((RELS, BN, FS))] * NS2
              + [_relblk((RELS, BN, CNTW)), _full((RELS, HID, HID)),
                 pl.BlockSpec((1, 1, BN), lambda i: (i, 0, 0)),
                 _full((HID, NLB)), _full((1, NLB))]),
    out_specs=_full((G, NLB)),
    out_shape=jax.ShapeDtypeStruct((G, NLB), jnp.float32),
    scratch_shapes=[pltpu.VMEM((G, HID), jnp.float32),
                    pltpu.VMEM((G, 128), jnp.float32)],
)

_sc_cache = {}


def _get_sc():
    if "sc" not in _sc_cache:
        _sc_cache["sc"] = (_make_sc1(), _make_sc2())
    return _sc_cache["sc"]


def kernel(x, edge_index, edge_type, batch, shape_emb, color_emb, pos_emb,
           W1, W1_root, b1, W2, W2_root, b2, cls_W, cls_b):
    pad = EPAD - E
    src_p = jnp.concatenate(
        [edge_index[0], jnp.zeros((pad,), jnp.int32)]).reshape(NSUB, NB, BATCH)
    dst_p = jnp.concatenate(
        [edge_index[1], jnp.full((pad,), N, jnp.int32)]).reshape(NSUB, NB, BATCH)
    typ_p = jnp.concatenate(
        [edge_type, jnp.zeros((pad,), jnp.int32)]).reshape(NSUB, NB, BATCH)

    ones8 = jnp.ones((BATCH, CNTW), jnp.float32)
    zerosF = jnp.zeros((CHUNK, FS), jnp.float32)
    zeros8 = jnp.zeros((CHUNK, CNTW), jnp.float32)

    _sc1, _sc2 = _get_sc()
    f = _t1(x, shape_emb, color_emb, pos_emb)

    sc1_out = _sc1(*f, src_p, dst_p, typ_p, ones8, zerosF, zeros8)
    base1 = _t2a(*f, W1_root, b1.reshape(1, HID))
    a = sc1_out[:NS1]
    cnt, offs = sc1_out[NS1], sc1_out[NS1 + 1]
    a_r = [ai.reshape(RELS, NP2, FS) for ai in a]
    cnt_r = cnt.reshape(RELS, NP2, CNTW)

    h = _t2(base1, *a_r, cnt_r, W1)

    g = _sc2(*h, src_p, offs, zerosF)
    base2 = _t3a(*h, W2_root, b2.reshape(1, HID))
    g_r = [gi.reshape(RELS, NP2, FS) for gi in g]

    bat3 = batch.reshape(NBLK, 1, BN)
    out = _t3(base2, *g_r, cnt_r, W2, bat3, cls_W, cls_b.reshape(1, NLB))
    return out
